# baseline (speedup 1.0000x reference)
import functools

import jax
import jax.numpy as jnp
from jax import lax
from jax.experimental import pallas as pl
from jax.experimental.pallas import tpu as pltpu
from jax.experimental.pallas import tpu_sc as plsc

_N = 10000
_E = 320000
_D = 128
_G = 64
_EPS = 1e-5

_NC = 2
_NS = 16
_NW = _NC * _NS
_FPW = _D // _NW
_ECH = 4000


def _wid():
    return lax.axis_index("s") * _NC + lax.axis_index("c")


def _deg_body(dst_hbm, out_hbm, dst_v, acc_v):
    w = _wid()
    epw = _E // _NW
    pltpu.sync_copy(dst_hbm.at[pl.ds(w * epw, epw)], dst_v)

    @functools.partial(plsc.parallel_loop, 0, _N // 16, unroll=8)
    def zero(i):
        acc_v[pl.ds(i * 16, 16)] = jnp.zeros((16,), jnp.float32)

    ones = jnp.ones((16,), jnp.float32)

    @functools.partial(plsc.parallel_loop, 0, epw // 16, unroll=8)
    def body(i):
        d16 = dst_v[pl.ds(i * 16, 16)]
        plsc.addupdate_scatter(acc_v, [d16], ones)
    pltpu.sync_copy(acc_v, out_hbm.at[w])


def _degrees(dst):
    mesh = plsc.VectorSubcoreMesh(core_axis_name="c", subcore_axis_name="s")
    f = pl.kernel(
        _deg_body,
        out_type=jax.ShapeDtypeStruct((_NW, _N), jnp.float32),
        mesh=mesh,
        scratch_types=[
            pltpu.VMEM((_E // _NW,), jnp.int32),
            pltpu.VMEM((_N,), jnp.float32),
        ],
        compiler_params=pltpu.CompilerParams(needs_layout_passes=False),
    )
    return f(dst)


def _agg_body(gt_hbm, pk_hbm, out_hbm, g_v, acc_v, pk_v, sem):
    w = _wid()
    fbase = w * _FPW
    nch = _E // _ECH
    cpg = pltpu.async_copy(gt_hbm.at[pl.ds(fbase, _FPW)], g_v, sem)
    cpa = pltpu.async_copy(gt_hbm.at[pl.ds(fbase, _FPW)], acc_v, sem)
    pltpu.async_copy(pk_hbm.at[0], pk_v.at[0], sem)
    cpg.wait()
    cpa.wait()

    fvecs = [jnp.full((16,), f, jnp.int32) for f in range(_FPW)]

    def chunk(c, carry):
        b = lax.rem(c, 2)
        pltpu.make_async_copy(pk_hbm.at[c], pk_v.at[b], sem).wait()

        @pl.when(c + 1 < nch)
        def _prefetch():
            pltpu.async_copy(pk_hbm.at[c + 1], pk_v.at[1 - b], sem)

        @functools.partial(plsc.parallel_loop, 0, _ECH // 16, unroll=1)
        def inner(i):
            p16 = pk_v[b, pl.ds(i * 16, 16)]
            s16, d16 = plsc.unpack(plsc.bitcast(p16, jnp.int16),
                                   format=plsc.PackFormat.INTERLEAVED)
            for f in range(_FPW):
                v = plsc.load_gather(g_v, [fvecs[f], s16])
                plsc.addupdate_scatter(acc_v, [fvecs[f], d16], v)
        return carry
    lax.fori_loop(0, nch, chunk, 0)
    pltpu.sync_copy(acc_v, out_hbm.at[pl.ds(fbase, _FPW)])


def _aggregate(gt, packed):
    mesh = plsc.VectorSubcoreMesh(core_axis_name="c", subcore_axis_name="s")
    f = pl.kernel(
        _agg_body,
        out_type=jax.ShapeDtypeStruct((_D, _N), jnp.float32),
        mesh=mesh,
        scratch_types=[
            pltpu.VMEM((_FPW, _N), jnp.float32),
            pltpu.VMEM((_FPW, _N), jnp.float32),
            pltpu.VMEM((2, _ECH), jnp.int32),
            pltpu.SemaphoreType.DMA,
        ],
        compiler_params=pltpu.CompilerParams(needs_layout_passes=False),
    )
    return f(gt, packed.reshape(_E // _ECH, _ECH))


def _tc1_body(x_ref, w1_ref, degp_ref, src_ref, dst_ref,
              g1t_ref, ds_ref, pk_ref):
    deg = 1.0 + jnp.sum(degp_ref[...], axis=0, keepdims=True)
    ds = lax.rsqrt(deg)
    h = lax.dot_general(w1_ref[...], x_ref[...], (((0,), (1,)), ((), ())),
                        preferred_element_type=jnp.float32)
    g1t_ref[...] = h * ds
    ds_ref[...] = ds
    pk_ref[...] = src_ref[...] | (dst_ref[...] << 16)


def _tc1(x, W1, degp, src, dst):
    _R = _E // 512
    g1t, ds, pk = pl.pallas_call(
        _tc1_body,
        out_shape=(jax.ShapeDtypeStruct((_D, _N), jnp.float32),
                   jax.ShapeDtypeStruct((1, _N), jnp.float32),
                   jax.ShapeDtypeStruct((_R, 512), jnp.int32)),
    )(x, W1, degp, src.reshape(_R, 512), dst.reshape(_R, 512))
    return g1t, ds, pk.reshape(_E)


def _bn_relu(o, gam, bet):
    mean = jnp.mean(o, axis=1, keepdims=True)
    var = jnp.mean((o - mean) ** 2, axis=1, keepdims=True)
    o = (o - mean) * lax.rsqrt(var + _EPS) * gam + bet
    return jnp.maximum(o, 0.0)


def _tc2_body(s1t_ref, ds_ref, b1_ref, gam_ref, bet_ref, w2_ref, g2t_ref):
    ds = ds_ref[...]
    o = s1t_ref[...] * ds + b1_ref[...]
    o = _bn_relu(o, gam_ref[...], bet_ref[...])
    g2t_ref[...] = lax.dot_general(
        w2_ref[...], o, (((0,), (0,)), ((), ())),
        preferred_element_type=jnp.float32) * ds


def _tc2(s1t, ds, b1, gam, bet, W2):
    return pl.pallas_call(
        _tc2_body,
        out_shape=jax.ShapeDtypeStruct((_D, _N), jnp.float32),
    )(s1t, ds, b1, gam, bet, W2)


def _tc3_body(s2t_ref, ds_ref, b2_ref, gam_ref, bet_ref, batch_ref,
              wc_ref, bc_ref, out_ref):
    o = s2t_ref[...] * ds_ref[...] + b2_ref[...]
    h = _bn_relu(o, gam_ref[...], bet_ref[...])
    gid = lax.broadcasted_iota(jnp.int32, (_G, _N), 0)
    sel = (batch_ref[...] == gid).astype(jnp.float32)
    sums = lax.dot_general(sel, h, (((1,), (1,)), ((), ())),
                           preferred_element_type=jnp.float32)
    counts = jnp.sum(sel, axis=1, keepdims=True)
    pooled = sums / jnp.maximum(counts, 1.0)
    out_ref[...] = jnp.dot(pooled, wc_ref[...],
                           preferred_element_type=jnp.float32) + bc_ref[...]


def _tc3(s2t, ds, b2, gam, bet, batch, Wc, bc):
    return pl.pallas_call(
        _tc3_body,
        out_shape=jax.ShapeDtypeStruct((_G, Wc.shape[1]), jnp.float32),
    )(s2t, ds, b2, gam, bet, batch, Wc, bc)


def kernel(x, edge_index, batch, W1, b1, gamma1, beta1,
           W2, b2, gamma2, beta2, Wc, bc):
    src = edge_index[0].astype(jnp.int32)
    dst = edge_index[1].astype(jnp.int32)
    batch_i = batch.astype(jnp.int32).reshape(1, _N)

    degp = _degrees(dst)
    g1t, ds, packed = _tc1(x, W1, degp, src, dst)
    s1t = _aggregate(g1t, packed)
    g2t = _tc2(s1t, ds, b1.reshape(_D, 1), gamma1.reshape(_D, 1),
               beta1.reshape(_D, 1), W2)
    s2t = _aggregate(g2t, packed)
    return _tc3(s2t, ds, b2.reshape(_D, 1), gamma2.reshape(_D, 1),
                beta2.reshape(_D, 1), batch_i, Wc, bc.reshape(1, -1))


# speedup vs baseline: 1.0132x; 1.0132x over previous
import functools

import jax
import jax.numpy as jnp
from jax import lax
from jax.experimental import pallas as pl
from jax.experimental.pallas import tpu as pltpu
from jax.experimental.pallas import tpu_sc as plsc

_N = 10000
_E = 320000
_D = 128
_G = 64
_EPS = 1e-5

_NC = 2
_NS = 16
_NW = _NC * _NS
_FPW = _D // _NW
_ECH = 4000


def _wid():
    return lax.axis_index("s") * _NC + lax.axis_index("c")


def _deg_body(dst_hbm, out_hbm, dst_v, acc_v):
    w = _wid()
    epw = _E // _NW
    pltpu.sync_copy(dst_hbm.at[pl.ds(w * epw, epw)], dst_v)

    @functools.partial(plsc.parallel_loop, 0, _N // 16, unroll=8)
    def zero(i):
        acc_v[pl.ds(i * 16, 16)] = jnp.zeros((16,), jnp.float32)

    ones = jnp.ones((16,), jnp.float32)

    @functools.partial(plsc.parallel_loop, 0, epw // 16, unroll=8)
    def body(i):
        d16 = dst_v[pl.ds(i * 16, 16)]
        plsc.addupdate_scatter(acc_v, [d16], ones)
    pltpu.sync_copy(acc_v, out_hbm.at[w])


def _degrees(dst):
    mesh = plsc.VectorSubcoreMesh(core_axis_name="c", subcore_axis_name="s")
    f = pl.kernel(
        _deg_body,
        out_type=jax.ShapeDtypeStruct((_NW, _N), jnp.float32),
        mesh=mesh,
        scratch_types=[
            pltpu.VMEM((_E // _NW,), jnp.int32),
            pltpu.VMEM((_N,), jnp.float32),
        ],
        compiler_params=pltpu.CompilerParams(needs_layout_passes=False),
    )
    return f(dst)


def _agg_body(gp_hbm, pk_hbm, out_hbm, g_v, acc_v, pk_v, sem):
    w = _wid()
    npp = _FPW // 2
    nch = _E // _ECH
    cpg = pltpu.async_copy(gp_hbm.at[pl.ds(w * npp, npp)], g_v, sem)
    pltpu.async_copy(pk_hbm.at[0], pk_v.at[0], sem)
    cpg.wait()

    pvecs = [jnp.full((16,), p, jnp.int32) for p in range(npp)]
    rvecs = [jnp.full((16,), r, jnp.int32) for r in range(_FPW)]

    for p in range(npp):
        @functools.partial(plsc.parallel_loop, 0, _N // 16, unroll=1)
        def init(i):
            wv = g_v[p, pl.ds(i * 16, 16)]
            vlo, vhi = plsc.unpack(plsc.bitcast(wv, jnp.bfloat16),
                                   format=plsc.PackFormat.INTERLEAVED)
            acc_v[2 * p, pl.ds(i * 16, 16)] = vlo
            acc_v[2 * p + 1, pl.ds(i * 16, 16)] = vhi

    def chunk(c, carry):
        b = lax.rem(c, 2)
        pltpu.make_async_copy(pk_hbm.at[c], pk_v.at[b], sem).wait()

        @pl.when(c + 1 < nch)
        def _prefetch():
            pltpu.async_copy(pk_hbm.at[c + 1], pk_v.at[1 - b], sem)

        @functools.partial(plsc.parallel_loop, 0, _ECH // 16, unroll=1)
        def inner(i):
            p16 = pk_v[b, pl.ds(i * 16, 16)]
            s16, d16 = plsc.unpack(plsc.bitcast(p16, jnp.int16),
                                   format=plsc.PackFormat.INTERLEAVED)
            for p in range(npp):
                wv = plsc.load_gather(g_v, [pvecs[p], s16])
                vlo, vhi = plsc.unpack(plsc.bitcast(wv, jnp.bfloat16),
                                       format=plsc.PackFormat.INTERLEAVED)
                plsc.addupdate_scatter(acc_v, [rvecs[2 * p], d16], vlo)
                plsc.addupdate_scatter(acc_v, [rvecs[2 * p + 1], d16], vhi)
        return carry
    lax.fori_loop(0, nch, chunk, 0)
    pltpu.sync_copy(acc_v, out_hbm.at[pl.ds(w * _FPW, _FPW)])


def _aggregate(gp, packed):
    mesh = plsc.VectorSubcoreMesh(core_axis_name="c", subcore_axis_name="s")
    f = pl.kernel(
        _agg_body,
        out_type=jax.ShapeDtypeStruct((_D, _N), jnp.float32),
        mesh=mesh,
        scratch_types=[
            pltpu.VMEM((_FPW // 2, _N), jnp.int32),
            pltpu.VMEM((_FPW, _N), jnp.float32),
            pltpu.VMEM((2, _ECH), jnp.int32),
            pltpu.SemaphoreType.DMA,
        ],
        compiler_params=pltpu.CompilerParams(needs_layout_passes=False),
    )
    return f(gp, packed.reshape(_E // _ECH, _ECH))


def _pack_pairs(g_perm):
    he = g_perm[: _D // 2].astype(jnp.bfloat16)
    ho = g_perm[_D // 2:].astype(jnp.bfloat16)
    e32 = lax.bitcast_convert_type(he, jnp.uint16).astype(jnp.int32)
    o32 = lax.bitcast_convert_type(ho, jnp.uint16).astype(jnp.int32)
    return e32 | lax.shift_left(o32, 16)


def _tc1_body(x_ref, w1p_ref, degp_ref, src_ref, dst_ref,
              gp1_ref, ds_ref, pk_ref):
    deg = 1.0 + jnp.sum(degp_ref[...], axis=0, keepdims=True)
    ds = lax.rsqrt(deg)
    h = lax.dot_general(w1p_ref[...], x_ref[...], (((0,), (1,)), ((), ())),
                        preferred_element_type=jnp.float32)
    gp1_ref[...] = _pack_pairs(h * ds)
    ds_ref[...] = ds
    pk_ref[...] = src_ref[...] | (dst_ref[...] << 16)


def _tc1(x, W1p, degp, src, dst):
    _R = _E // 512
    gp1, ds, pk = pl.pallas_call(
        _tc1_body,
        out_shape=(jax.ShapeDtypeStruct((_D // 2, _N), jnp.int32),
                   jax.ShapeDtypeStruct((1, _N), jnp.float32),
                   jax.ShapeDtypeStruct((_R, 512), jnp.int32)),
    )(x, W1p, degp, src.reshape(_R, 512), dst.reshape(_R, 512))
    return gp1, ds, pk.reshape(_E)


def _bn_relu(o, gam, bet):
    mean = jnp.mean(o, axis=1, keepdims=True)
    var = jnp.mean((o - mean) ** 2, axis=1, keepdims=True)
    o = (o - mean) * lax.rsqrt(var + _EPS) * gam + bet
    return jnp.maximum(o, 0.0)


def _tc2_body(s1t_ref, ds_ref, b1_ref, gam_ref, bet_ref, w2p_ref, gp2_ref):
    ds = ds_ref[...]
    o = s1t_ref[...] * ds + b1_ref[...]
    o = _bn_relu(o, gam_ref[...], bet_ref[...])
    g2 = lax.dot_general(
        w2p_ref[...], o, (((0,), (0,)), ((), ())),
        preferred_element_type=jnp.float32) * ds
    gp2_ref[...] = _pack_pairs(g2)


def _tc2(s1t, ds, b1, gam, bet, W2p):
    return pl.pallas_call(
        _tc2_body,
        out_shape=jax.ShapeDtypeStruct((_D // 2, _N), jnp.int32),
    )(s1t, ds, b1, gam, bet, W2p)


def _tc3_body(s2t_ref, ds_ref, b2_ref, gam_ref, bet_ref, batch_ref,
              wc_ref, bc_ref, out_ref):
    o = s2t_ref[...] * ds_ref[...] + b2_ref[...]
    h = _bn_relu(o, gam_ref[...], bet_ref[...])
    gid = lax.broadcasted_iota(jnp.int32, (_G, _N), 0)
    sel = (batch_ref[...] == gid).astype(jnp.float32)
    sums = lax.dot_general(sel, h, (((1,), (1,)), ((), ())),
                           preferred_element_type=jnp.float32)
    counts = jnp.sum(sel, axis=1, keepdims=True)
    pooled = sums / jnp.maximum(counts, 1.0)
    out_ref[...] = jnp.dot(pooled, wc_ref[...],
                           preferred_element_type=jnp.float32) + bc_ref[...]


def _tc3(s2t, ds, b2, gam, bet, batch, Wc, bc):
    return pl.pallas_call(
        _tc3_body,
        out_shape=jax.ShapeDtypeStruct((_G, Wc.shape[1]), jnp.float32),
    )(s2t, ds, b2, gam, bet, batch, Wc, bc)


def kernel(x, edge_index, batch, W1, b1, gamma1, beta1,
           W2, b2, gamma2, beta2, Wc, bc):
    src = edge_index[0].astype(jnp.int32)
    dst = edge_index[1].astype(jnp.int32)
    batch_i = batch.astype(jnp.int32).reshape(1, _N)

    W1p = jnp.concatenate([W1[:, 0::2], W1[:, 1::2]], axis=1)
    W2p = jnp.concatenate([W2[:, 0::2], W2[:, 1::2]], axis=1)

    degp = _degrees(dst)
    gp1, ds, packed = _tc1(x, W1p, degp, src, dst)
    s1t = _aggregate(gp1, packed)
    g2t = _tc2(s1t, ds, b1.reshape(_D, 1), gamma1.reshape(_D, 1),
               beta1.reshape(_D, 1), W2p)
    s2t = _aggregate(g2t, packed)
    return _tc3(s2t, ds, b2.reshape(_D, 1), gamma2.reshape(_D, 1),
                beta2.reshape(_D, 1), batch_i, Wc, bc.reshape(1, -1))
